# Initial kernel scaffold; baseline (speedup 1.0000x reference)
import jax
import jax.numpy as jnp
from jax.experimental import pallas as pl


def kernel(keys, queries):
    raise NotImplementedError("write your pallas kernel here")


# speedup vs baseline: 4.7492x; 4.7492x over previous
import functools

import jax
import jax.numpy as jnp
from jax.experimental import pallas as pl

K_NN = 10


def _knn_block(ksq_ref, qsq_ref, keys_ref, q_ref, idx_ref, dist_ref, *, k_nn):
    k_blk = keys_ref[...]
    q_all = q_ref[...]
    dot = jax.lax.dot_general(
        k_blk, q_all,
        dimension_numbers=(((1,), (1,)), ((), ())),
        preferred_element_type=jnp.float32,
        precision=jax.lax.Precision.DEFAULT,
    )
    d2 = (ksq_ref[...] + qsq_ref[...]) - 2.0 * dot
    d2 = jnp.maximum(d2, 0.0)

    tr, m = d2.shape
    iota = jax.lax.broadcasted_iota(jnp.int32, (tr, m), 1)
    cur = d2
    vals = []
    idxs = []
    for _ in range(k_nn):
        mn = jnp.min(cur, axis=1, keepdims=True)
        am = jnp.min(jnp.where(cur == mn, iota, m), axis=1,
                     keepdims=True)
        vals.append(mn)
        idxs.append(am)
        cur = jnp.where(iota == am, jnp.inf, cur)
    idx_ref[...] = jnp.concatenate(idxs, axis=1)
    dist_ref[...] = jnp.concatenate(vals, axis=1)


@functools.partial(jax.jit, static_argnames=())
def kernel(keys, queries):
    n, d = keys.shape
    m, _ = queries.shape
    ksq = jnp.sum(keys * keys, axis=1, keepdims=True)
    qsq = jnp.sum(queries * queries, axis=1, keepdims=True)
    qsq_row = qsq.T

    tile_r = 128 if n % 128 == 0 else n
    grid = (n // tile_r,)

    idx_out, dist_out = pl.pallas_call(
        functools.partial(_knn_block, k_nn=K_NN),
        grid=grid,
        in_specs=[
            pl.BlockSpec((tile_r, 1), lambda i: (i, 0)),
            pl.BlockSpec((1, m), lambda i: (0, 0)),
            pl.BlockSpec((tile_r, d), lambda i: (i, 0)),
            pl.BlockSpec((m, d), lambda i: (0, 0)),
        ],
        out_specs=[
            pl.BlockSpec((tile_r, K_NN), lambda i: (i, 0)),
            pl.BlockSpec((tile_r, K_NN), lambda i: (i, 0)),
        ],
        out_shape=[
            jax.ShapeDtypeStruct((n, K_NN), jnp.int32),
            jax.ShapeDtypeStruct((n, K_NN), jnp.float32),
        ],
    )(ksq, qsq_row, keys, queries)
    return idx_out, dist_out
